# baseline (speedup 1.0000x reference)
import functools

import jax
import jax.numpy as jnp
from jax import lax
from jax.experimental import pallas as pl
from jax.experimental.pallas import tpu as pltpu
from jax.experimental.pallas import tpu_sc as plsc

N_CORES = 2
N_SUB = 16
NW = N_CORES * N_SUB
CHUNK = 128
NBUF = 2
PAD_ROWS = 240


def _sc_segment_sum(h, e_flat, n_pad, n_group):
    D = h.shape[1]
    rows_per_sub = n_pad // N_SUB
    n_init = rows_per_sub // CHUNK

    mesh = plsc.VectorSubcoreMesh(core_axis_name="c", subcore_axis_name="s")

    @functools.partial(
        pl.kernel,
        out_type=(
            jax.ShapeDtypeStruct((N_CORES, n_pad, D), jnp.float32),
            jax.ShapeDtypeStruct((N_CORES * n_pad,), jnp.float32),
        ),
        mesh=mesh,
        scratch_types=[
            pltpu.VMEM((2, NBUF, CHUNK), jnp.int32),
            pltpu.VMEM((2, NBUF, CHUNK), jnp.int32),
            pltpu.VMEM((NBUF, CHUNK, D), jnp.float32),
            pltpu.VMEM((CHUNK,), jnp.float32),
            pltpu.VMEM((CHUNK,), jnp.float32),
            pltpu.VMEM_SHARED((n_pad, D), jnp.float32),
            pltpu.VMEM_SHARED((n_pad,), jnp.float32),
            [pltpu.SemaphoreType.DMA] * NBUF,
            [pltpu.SemaphoreType.DMA] * 2,
            pltpu.SemaphoreType.DMA,
        ],
    )
    def k(h_hbm, e_hbm, acc_hbm, deg_hbm,
          src_v, dst_v, rows_v, ones_v, zv, acc_sh, deg_sh, sem_g, sem_i,
          sem_s):
        c = lax.axis_index("c")
        s = lax.axis_index("s")
        wid = s * N_CORES + c

        zero16 = jnp.zeros((16,), jnp.float32)
        for i in range(CHUNK // 16):
            ones_v[pl.ds(i * 16, 16)] = jnp.ones((16,), jnp.float32)
            zv[pl.ds(i * 16, 16)] = zero16

        def zrow(r, carry):
            for j in range(D // 16):
                rows_v[0, r, pl.ds(j * 16, 16)] = zero16
            return carry

        lax.fori_loop(0, CHUNK, zrow, 0)

        r0 = s * rows_per_sub
        for t in range(n_init):
            pltpu.sync_copy(rows_v.at[0], acc_sh.at[pl.ds(r0 + t * CHUNK, CHUNK)])
            pltpu.sync_copy(zv, deg_sh.at[pl.ds(r0 + t * CHUNK, CHUNK)])

        plsc.subcore_barrier()

        ew = n_group * NBUF * CHUNK
        e_half = NW * ew
        src0 = wid * ew
        dst0 = e_half + wid * ew

        for bi in range(NBUF):
            pltpu.sync_copy(e_hbm.at[pl.ds(src0 + bi * CHUNK, CHUNK)],
                            src_v.at[0, bi])
            pltpu.sync_copy(e_hbm.at[pl.ds(dst0 + bi * CHUNK, CHUNK)],
                            dst_v.at[0, bi])

        def group(g, p):
            gn = jnp.minimum(g + 1, n_group - 1) * (NBUF * CHUNK)
            ip = []
            for bi in range(NBUF):
                ip.append(pltpu.async_copy(
                    e_hbm.at[pl.ds(src0 + gn + bi * CHUNK, CHUNK)],
                    src_v.at[1 - p, bi], sem_i[0]))
                ip.append(pltpu.async_copy(
                    e_hbm.at[pl.ds(dst0 + gn + bi * CHUNK, CHUNK)],
                    dst_v.at[1 - p, bi], sem_i[1]))
            gathers = [
                pltpu.async_copy(h_hbm.at[src_v.at[p, bi]],
                                 rows_v.at[bi], sem_g[bi])
                for bi in range(NBUF)
            ]
            scat = []
            for bi in range(NBUF):
                gathers[bi].wait()
                scat.append(
                    pltpu.async_copy(rows_v.at[bi],
                                     acc_sh.at[dst_v.at[p, bi]],
                                     sem_s, add=True))
                scat.append(
                    pltpu.async_copy(ones_v,
                                     deg_sh.at[dst_v.at[p, bi]],
                                     sem_s, add=True))
            for d in scat:
                d.wait()
            for d in ip:
                d.wait()
            return 1 - p

        lax.fori_loop(0, n_group, group, 0)

        plsc.subcore_barrier()

        pltpu.sync_copy(acc_sh.at[pl.ds(r0, rows_per_sub)],
                        acc_hbm.at[c, pl.ds(r0, rows_per_sub)])
        pltpu.sync_copy(deg_sh.at[pl.ds(r0, rows_per_sub)],
                        deg_hbm.at[pl.ds(c * n_pad + r0, rows_per_sub)])

    return k(h, e_flat)


def _proj_kernel(x_ref, w_ref, o_ref):
    o_ref[...] = jnp.dot(x_ref[...], w_ref[...],
                         preferred_element_type=jnp.float32)


def _final_kernel(p_ref, d_ref, h_ref, w_ref, b_ref, o_ref):
    agg = p_ref[0] + p_ref[1] + h_ref[...]
    deg = d_ref[:, 0] + d_ref[:, 1] + 1.0
    agg = agg / deg[:, None]
    o_ref[...] = jnp.maximum(
        jnp.dot(agg, w_ref[...], preferred_element_type=jnp.float32)
        + b_ref[...], 0.0)


def kernel(x, edge_index, W_proj, W_out, b):
    N, D = x.shape
    E = edge_index.shape[1]

    step = CHUNK * NBUF
    epw = E // NW
    ew = ((epw + step - 1) // step) * step
    n_group = ew // step
    e_pad = NW * ew
    n_pad = N + PAD_ROWS

    npad_e = e_pad - E
    pad_src = (jnp.arange(npad_e, dtype=jnp.int32) * 37) % N
    pad_dst = N + (jnp.arange(npad_e, dtype=jnp.int32) % PAD_ROWS)
    e_flat = jnp.concatenate(
        [edge_index[0], pad_src, edge_index[1], pad_dst])

    blk = 1000
    grid = N // blk
    h = pl.pallas_call(
        _proj_kernel,
        grid=(grid,),
        in_specs=[pl.BlockSpec((blk, D), lambda i: (i, 0)),
                  pl.BlockSpec((D, D), lambda i: (0, 0))],
        out_specs=pl.BlockSpec((blk, D), lambda i: (i, 0)),
        out_shape=jax.ShapeDtypeStruct((N, D), jnp.float32),
    )(x, W_proj)

    acc, deg = _sc_segment_sum(h, e_flat, n_pad, n_group)
    deg = deg.reshape(N_CORES, n_pad)[:, :N].T

    out = pl.pallas_call(
        _final_kernel,
        grid=(grid,),
        in_specs=[
            pl.BlockSpec((N_CORES, blk, D), lambda i: (0, i, 0)),
            pl.BlockSpec((blk, N_CORES), lambda i: (i, 0)),
            pl.BlockSpec((blk, D), lambda i: (i, 0)),
            pl.BlockSpec((D, D), lambda i: (0, 0)),
            pl.BlockSpec((1, D), lambda i: (0, 0)),
        ],
        out_specs=pl.BlockSpec((blk, D), lambda i: (i, 0)),
        out_shape=jax.ShapeDtypeStruct((N, D), jnp.float32),
    )(acc, deg, h, W_out, b.reshape(1, D))
    return out


# speedup vs baseline: 1.0648x; 1.0648x over previous
import functools

import jax
import jax.numpy as jnp
from jax import lax
from jax.experimental import pallas as pl
from jax.experimental.pallas import tpu as pltpu
from jax.experimental.pallas import tpu_sc as plsc

N_CORES = 2
N_SUB = 16
NW = N_CORES * N_SUB
CHUNK = 128
NBUF = 2
PAD_ROWS = 240


def _sc_segment_sum(h, e_flat, n_pad, n_group):
    D = h.shape[1]
    rows_per_sub = n_pad // N_SUB
    n_init = rows_per_sub // CHUNK

    mesh = plsc.VectorSubcoreMesh(core_axis_name="c", subcore_axis_name="s")

    @functools.partial(
        pl.kernel,
        out_type=(
            jax.ShapeDtypeStruct((N_CORES, n_pad, D), jnp.float32),
            jax.ShapeDtypeStruct((N_CORES * n_pad,), jnp.float32),
        ),
        mesh=mesh,
        scratch_types=[
            pltpu.VMEM((2, NBUF, CHUNK), jnp.int32),
            pltpu.VMEM((2, NBUF, CHUNK), jnp.int32),
            pltpu.VMEM((NBUF, CHUNK, D), jnp.float32),
            pltpu.VMEM((CHUNK,), jnp.float32),
            pltpu.VMEM((CHUNK,), jnp.float32),
            pltpu.VMEM_SHARED((n_pad, D), jnp.float32),
            pltpu.VMEM_SHARED((n_pad,), jnp.float32),
            [pltpu.SemaphoreType.DMA] * NBUF,
            [pltpu.SemaphoreType.DMA] * 2,
            pltpu.SemaphoreType.DMA,
        ],
    )
    def k(h_hbm, e_hbm, acc_hbm, deg_hbm,
          src_v, dst_v, rows_v, ones_v, zv, acc_sh, deg_sh, sem_g, sem_i,
          sem_s):
        c = lax.axis_index("c")
        s = lax.axis_index("s")
        wid = s * N_CORES + c

        zero16 = jnp.zeros((16,), jnp.float32)
        for i in range(CHUNK // 16):
            ones_v[pl.ds(i * 16, 16)] = jnp.ones((16,), jnp.float32)
            zv[pl.ds(i * 16, 16)] = zero16

        def zrow(r, carry):
            for j in range(D // 16):
                rows_v[0, r, pl.ds(j * 16, 16)] = zero16
            return carry

        lax.fori_loop(0, CHUNK, zrow, 0)

        r0 = s * rows_per_sub
        for t in range(n_init):
            pltpu.sync_copy(rows_v.at[0], acc_sh.at[pl.ds(r0 + t * CHUNK, CHUNK)])
            pltpu.sync_copy(zv, deg_sh.at[pl.ds(r0 + t * CHUNK, CHUNK)])

        plsc.subcore_barrier()

        ew = n_group * NBUF * CHUNK
        e_half = NW * ew
        src0 = wid * ew
        dst0 = e_half + wid * ew

        for bi in range(NBUF):
            pltpu.sync_copy(e_hbm.at[pl.ds(src0 + bi * CHUNK, CHUNK)],
                            src_v.at[0, bi])
            pltpu.sync_copy(e_hbm.at[pl.ds(dst0 + bi * CHUNK, CHUNK)],
                            dst_v.at[0, bi])

        def group(g, p):
            gn = jnp.minimum(g + 1, n_group - 1) * (NBUF * CHUNK)
            ip = []
            for bi in range(NBUF):
                ip.append(pltpu.async_copy(
                    e_hbm.at[pl.ds(src0 + gn + bi * CHUNK, CHUNK)],
                    src_v.at[1 - p, bi], sem_i[0]))
                ip.append(pltpu.async_copy(
                    e_hbm.at[pl.ds(dst0 + gn + bi * CHUNK, CHUNK)],
                    dst_v.at[1 - p, bi], sem_i[1]))
            gathers = [
                pltpu.async_copy(h_hbm.at[src_v.at[p, bi]],
                                 rows_v.at[bi], sem_g[bi])
                for bi in range(NBUF)
            ]
            scat = []
            for bi in range(NBUF):
                gathers[bi].wait()
                scat.append(
                    pltpu.async_copy(rows_v.at[bi],
                                     acc_sh.at[dst_v.at[p, bi]],
                                     sem_s, add=True))
                scat.append(
                    pltpu.async_copy(ones_v,
                                     deg_sh.at[dst_v.at[p, bi]],
                                     sem_s, add=True))
            for d in scat:
                d.wait()
            for d in ip:
                d.wait()
            return 1 - p

        lax.fori_loop(0, n_group, group, 0)

        plsc.subcore_barrier()

        pltpu.sync_copy(acc_sh.at[pl.ds(r0, rows_per_sub)],
                        acc_hbm.at[c, pl.ds(r0, rows_per_sub)])
        pltpu.sync_copy(deg_sh.at[pl.ds(r0, rows_per_sub)],
                        deg_hbm.at[pl.ds(c * n_pad + r0, rows_per_sub)])

    return k(h, e_flat)


def _edge_prep_kernel(e_ref, o_ref, *, E, npad_e, N, pad_rows):
    io = lax.iota(jnp.int32, npad_e)
    o_ref[pl.ds(0, E)] = e_ref[0, :]
    o_ref[pl.ds(E, npad_e)] = (io * 37) % N
    o_ref[pl.ds(E + npad_e, E)] = e_ref[1, :]
    o_ref[pl.ds(2 * E + npad_e, npad_e)] = N + (io % pad_rows)


def _proj_kernel(x_ref, w_ref, o_ref):
    o_ref[...] = jnp.dot(x_ref[...], w_ref[...],
                         preferred_element_type=jnp.float32)


def _final_kernel(p_ref, d_ref, h_ref, w_ref, b_ref, o_ref):
    agg = p_ref[0] + p_ref[1] + h_ref[...]
    deg = d_ref[:, 0] + d_ref[:, 1] + 1.0
    agg = agg / deg[:, None]
    o_ref[...] = jnp.maximum(
        jnp.dot(agg, w_ref[...], preferred_element_type=jnp.float32)
        + b_ref[...], 0.0)


def kernel(x, edge_index, W_proj, W_out, b):
    N, D = x.shape
    E = edge_index.shape[1]

    step = CHUNK * NBUF
    epw = E // NW
    ew = ((epw + step - 1) // step) * step
    n_group = ew // step
    e_pad = NW * ew
    n_pad = N + PAD_ROWS

    npad_e = e_pad - E
    e_flat = pl.pallas_call(
        functools.partial(_edge_prep_kernel, E=E, npad_e=npad_e, N=N,
                          pad_rows=PAD_ROWS),
        grid=(1,),
        in_specs=[pl.BlockSpec((2, E), lambda i: (0, 0))],
        out_specs=pl.BlockSpec((2 * e_pad,), lambda i: (0,)),
        out_shape=jax.ShapeDtypeStruct((2 * e_pad,), jnp.int32),
    )(edge_index)

    blk = 1000
    grid = N // blk
    h = pl.pallas_call(
        _proj_kernel,
        grid=(grid,),
        in_specs=[pl.BlockSpec((blk, D), lambda i: (i, 0)),
                  pl.BlockSpec((D, D), lambda i: (0, 0))],
        out_specs=pl.BlockSpec((blk, D), lambda i: (i, 0)),
        out_shape=jax.ShapeDtypeStruct((N, D), jnp.float32),
    )(x, W_proj)

    acc, deg = _sc_segment_sum(h, e_flat, n_pad, n_group)
    deg = deg.reshape(N_CORES, n_pad)[:, :N].T

    out = pl.pallas_call(
        _final_kernel,
        grid=(grid,),
        in_specs=[
            pl.BlockSpec((N_CORES, blk, D), lambda i: (0, i, 0)),
            pl.BlockSpec((blk, N_CORES), lambda i: (i, 0)),
            pl.BlockSpec((blk, D), lambda i: (i, 0)),
            pl.BlockSpec((D, D), lambda i: (0, 0)),
            pl.BlockSpec((1, D), lambda i: (0, 0)),
        ],
        out_specs=pl.BlockSpec((blk, D), lambda i: (i, 0)),
        out_shape=jax.ShapeDtypeStruct((N, D), jnp.float32),
    )(acc, deg, h, W_out, b.reshape(1, D))
    return out


# speedup vs baseline: 1.0937x; 1.0272x over previous
import functools

import jax
import jax.numpy as jnp
from jax import lax
from jax.experimental import pallas as pl
from jax.experimental.pallas import tpu as pltpu
from jax.experimental.pallas import tpu_sc as plsc

N_CORES = 2
N_SUB = 16
NW = N_CORES * N_SUB
CHUNK = 128
NBUF = 2
PAD_ROWS = 240


def _sc_segment_sum(h, e_flat, n_pad, n_group):
    D = h.shape[1]
    rows_per_sub = n_pad // N_SUB
    n_init = rows_per_sub // CHUNK

    mesh = plsc.VectorSubcoreMesh(core_axis_name="c", subcore_axis_name="s")

    @functools.partial(
        pl.kernel,
        out_type=(
            jax.ShapeDtypeStruct((N_CORES, n_pad, D), jnp.float32),
            jax.ShapeDtypeStruct((N_CORES * n_pad,), jnp.float32),
        ),
        mesh=mesh,
        scratch_types=[
            pltpu.VMEM((2, NBUF, CHUNK), jnp.int32),
            pltpu.VMEM((2, NBUF, CHUNK), jnp.int32),
            pltpu.VMEM((NBUF, CHUNK, D), jnp.float32),
            pltpu.VMEM((CHUNK,), jnp.float32),
            pltpu.VMEM((CHUNK,), jnp.float32),
            pltpu.VMEM_SHARED((n_pad, D), jnp.float32),
            pltpu.VMEM_SHARED((n_pad,), jnp.float32),
            [pltpu.SemaphoreType.DMA] * NBUF,
            [pltpu.SemaphoreType.DMA] * 2,
            pltpu.SemaphoreType.DMA,
        ],
    )
    def k(h_hbm, e_hbm, acc_hbm, deg_hbm,
          src_v, dst_v, rows_v, ones_v, zv, acc_sh, deg_sh, sem_g, sem_i,
          sem_s):
        c = lax.axis_index("c")
        s = lax.axis_index("s")
        wid = s * N_CORES + c

        zero16 = jnp.zeros((16,), jnp.float32)
        for i in range(CHUNK // 16):
            ones_v[pl.ds(i * 16, 16)] = jnp.ones((16,), jnp.float32)
            zv[pl.ds(i * 16, 16)] = zero16

        def zrow(r, carry):
            for j in range(D // 16):
                rows_v[0, r, pl.ds(j * 16, 16)] = zero16
            return carry

        lax.fori_loop(0, CHUNK, zrow, 0)

        r0 = s * rows_per_sub
        for t in range(n_init):
            pltpu.sync_copy(rows_v.at[0], acc_sh.at[pl.ds(r0 + t * CHUNK, CHUNK)])
            pltpu.sync_copy(zv, deg_sh.at[pl.ds(r0 + t * CHUNK, CHUNK)])

        plsc.subcore_barrier()

        ew = n_group * NBUF * CHUNK
        e_half = NW * ew
        src0 = wid * ew
        dst0 = e_half + wid * ew

        for bi in range(NBUF):
            pltpu.sync_copy(e_hbm.at[pl.ds(src0 + bi * CHUNK, CHUNK)],
                            src_v.at[0, bi])
            pltpu.sync_copy(e_hbm.at[pl.ds(dst0 + bi * CHUNK, CHUNK)],
                            dst_v.at[0, bi])

        def group(g, p):
            gn = jnp.minimum(g + 1, n_group - 1) * (NBUF * CHUNK)
            ip = []
            for bi in range(NBUF):
                ip.append(pltpu.async_copy(
                    e_hbm.at[pl.ds(src0 + gn + bi * CHUNK, CHUNK)],
                    src_v.at[1 - p, bi], sem_i[0]))
                ip.append(pltpu.async_copy(
                    e_hbm.at[pl.ds(dst0 + gn + bi * CHUNK, CHUNK)],
                    dst_v.at[1 - p, bi], sem_i[1]))
            gathers = [
                pltpu.async_copy(h_hbm.at[src_v.at[p, bi]],
                                 rows_v.at[bi], sem_g[bi])
                for bi in range(NBUF)
            ]
            scat = []
            for bi in range(NBUF):
                gathers[bi].wait()
                scat.append(
                    pltpu.async_copy(rows_v.at[bi],
                                     acc_sh.at[dst_v.at[p, bi]],
                                     sem_s, add=True))
                scat.append(
                    pltpu.async_copy(ones_v,
                                     deg_sh.at[dst_v.at[p, bi]],
                                     sem_s, add=True))
            for d in scat:
                d.wait()
            for d in ip:
                d.wait()
            return 1 - p

        lax.fori_loop(0, n_group, group, 0)

        plsc.subcore_barrier()

        pltpu.sync_copy(acc_sh.at[pl.ds(r0, rows_per_sub)],
                        acc_hbm.at[c, pl.ds(r0, rows_per_sub)])
        pltpu.sync_copy(deg_sh.at[pl.ds(r0, rows_per_sub)],
                        deg_hbm.at[pl.ds(c * n_pad + r0, rows_per_sub)])

    return k(h, e_flat)


def _edge_prep_kernel(e_ref, o_ref, *, E, npad_e, N, pad_rows):
    io = lax.iota(jnp.int32, npad_e)
    o_ref[pl.ds(0, E)] = e_ref[0, :]
    o_ref[pl.ds(E, npad_e)] = (io * 37) % N
    o_ref[pl.ds(E + npad_e, E)] = e_ref[1, :]
    o_ref[pl.ds(2 * E + npad_e, npad_e)] = N + (io % pad_rows)


def _proj_kernel(x_ref, w_ref, o_ref):
    o_ref[...] = jnp.dot(x_ref[...], w_ref[...],
                         preferred_element_type=jnp.float32)


def _final_kernel(p_ref, d_ref, h_ref, w_ref, b_ref, o_ref):
    agg = p_ref[0] + p_ref[1] + h_ref[...]
    deg = d_ref[:, 0] + d_ref[:, 1] + 1.0
    agg = agg / deg[:, None]
    o_ref[...] = jnp.maximum(
        jnp.dot(agg, w_ref[...], preferred_element_type=jnp.float32)
        + b_ref[...], 0.0)


def kernel(x, edge_index, W_proj, W_out, b):
    N, D = x.shape
    E = edge_index.shape[1]

    step = CHUNK * NBUF
    epw = E // NW
    ew = ((epw + step - 1) // step) * step
    n_group = ew // step
    e_pad = NW * ew
    n_pad = N + PAD_ROWS

    npad_e = e_pad - E
    e_flat = pl.pallas_call(
        functools.partial(_edge_prep_kernel, E=E, npad_e=npad_e, N=N,
                          pad_rows=PAD_ROWS),
        grid=(1,),
        in_specs=[pl.BlockSpec((2, E), lambda i: (0, 0))],
        out_specs=pl.BlockSpec((2 * e_pad,), lambda i: (0,)),
        out_shape=jax.ShapeDtypeStruct((2 * e_pad,), jnp.int32),
    )(edge_index)

    blk = 2000
    grid = N // blk
    h = pl.pallas_call(
        _proj_kernel,
        grid=(grid,),
        in_specs=[pl.BlockSpec((blk, D), lambda i: (i, 0)),
                  pl.BlockSpec((D, D), lambda i: (0, 0))],
        out_specs=pl.BlockSpec((blk, D), lambda i: (i, 0)),
        out_shape=jax.ShapeDtypeStruct((N, D), jnp.float32),
    )(x, W_proj)

    acc, deg = _sc_segment_sum(h, e_flat, n_pad, n_group)
    deg = deg.reshape(N_CORES, n_pad)[:, :N].T

    out = pl.pallas_call(
        _final_kernel,
        grid=(grid,),
        in_specs=[
            pl.BlockSpec((N_CORES, blk, D), lambda i: (0, i, 0)),
            pl.BlockSpec((blk, N_CORES), lambda i: (i, 0)),
            pl.BlockSpec((blk, D), lambda i: (i, 0)),
            pl.BlockSpec((D, D), lambda i: (0, 0)),
            pl.BlockSpec((1, D), lambda i: (0, 0)),
        ],
        out_specs=pl.BlockSpec((blk, D), lambda i: (i, 0)),
        out_shape=jax.ShapeDtypeStruct((N, D), jnp.float32),
    )(acc, deg, h, W_out, b.reshape(1, D))
    return out
